# baseline (speedup 1.0000x reference)
import functools

import jax
import jax.numpy as jnp
from jax import lax
from jax.experimental import pallas as pl
from jax.experimental.pallas import tpu as pltpu
from jax.experimental.pallas import tpu_sc as plsc

D = 64
TP = 128
B = 16384
NC = 2
NS = 16
NW = NC * NS
BPW = B // NW
CH = 128
NCH = BPW // CH
L = 16

_mesh = plsc.VectorSubcoreMesh(core_axis_name="c", subcore_axis_name="s")


@functools.partial(
    pl.kernel,
    mesh=_mesh,
    out_type=jax.ShapeDtypeStruct((D, B), jnp.float32),
    scratch_types=[
        pltpu.VMEM((BPW,), jnp.int32),
        pltpu.VMEM((BPW, TP), jnp.float32),
        pltpu.VMEM((D, BPW), jnp.float32),
        pltpu.SemaphoreType.DMA,
    ],
    compiler_params=pltpu.CompilerParams(
        use_tc_tiling_on_sc=True, needs_layout_passes=False
    ),
)
def _gather_kernel(idx_hbm, table_hbm, out_hbm, idx_v, rows_v, outb_v, sem):
    wid = lax.axis_index("s") * NC + lax.axis_index("c")
    base = wid * BPW
    pltpu.sync_copy(idx_hbm.at[pl.ds(base, BPW)], idx_v)
    copies = [
        pltpu.async_copy(
            table_hbm.at[idx_v.at[pl.ds(j * CH, CH)]],
            rows_v.at[pl.ds(j * CH, CH)],
            sem,
        )
        for j in range(NCH)
    ]
    for cp in copies:
        cp.wait()
    lane = lax.iota(jnp.int32, L)
    for bg in range(BPW // L):
        row_idx = lane + (bg * L)

        def body(c, _):
            col_idx = jnp.full((L,), c, jnp.int32)
            v = plsc.load_gather(rows_v, [row_idx, col_idx])
            outb_v[c, pl.ds(bg * L, L)] = v
            return _

        lax.fori_loop(0, D, body, None)
    pltpu.sync_copy(outb_v, out_hbm.at[:, pl.ds(base, BPW)])


def kernel(speaker_ids, table):
    tpad = jnp.pad(table, ((0, 0), (0, TP - D)))
    out_t = _gather_kernel(speaker_ids.astype(jnp.int32), tpad)
    return out_t.T


# speedup vs baseline: 1.1334x; 1.1334x over previous
import functools

import jax
import jax.numpy as jnp
from jax import lax
from jax.experimental import pallas as pl
from jax.experimental.pallas import tpu as pltpu
from jax.experimental.pallas import tpu_sc as plsc

D = 64
TP = 128
B = 16384
NC = 2
NS = 16
NW = NC * NS
BPW = B // NW
CH = 128
NCH = BPW // CH
L = 16

_mesh = plsc.VectorSubcoreMesh(core_axis_name="c", subcore_axis_name="s")


@functools.partial(
    pl.kernel,
    mesh=_mesh,
    out_type=jax.ShapeDtypeStruct((D, B), jnp.float32),
    scratch_types=[
        pltpu.VMEM((BPW,), jnp.int32),
        pltpu.VMEM((BPW, TP), jnp.float32),
        pltpu.VMEM((D, BPW), jnp.float32),
        pltpu.SemaphoreType.DMA,
    ],
    compiler_params=pltpu.CompilerParams(
        use_tc_tiling_on_sc=True, needs_layout_passes=False
    ),
)
def _gather_kernel(idx_hbm, table_hbm, out_hbm, idx_v, rows_v, outb_v, sem):
    wid = lax.axis_index("s") * NC + lax.axis_index("c")
    base = wid * BPW
    pltpu.sync_copy(idx_hbm.at[pl.ds(base, BPW)], idx_v)
    copies = [
        pltpu.async_copy(
            table_hbm.at[idx_v.at[pl.ds(j * CH, CH)]],
            rows_v.at[pl.ds(j * CH, CH)],
            sem,
        )
        for j in range(NCH)
    ]
    for cp in copies:
        cp.wait()
    lane = lax.iota(jnp.int32, L)

    @plsc.parallel_loop(0, D, unroll=4)
    def _transpose(c):
        col_idx = jnp.full((L,), c, jnp.int32)
        for bg in range(BPW // L):
            row_idx = lane + (bg * L)
            v = plsc.load_gather(rows_v, [row_idx, col_idx])
            outb_v[c, pl.ds(bg * L, L)] = v
    pltpu.sync_copy(outb_v, out_hbm.at[:, pl.ds(base, BPW)])


def kernel(speaker_ids, table):
    tpad = jnp.pad(table, ((0, 0), (0, TP - D)))
    out_t = _gather_kernel(speaker_ids.astype(jnp.int32), tpad)
    return out_t.T
